# Initial kernel scaffold; baseline (speedup 1.0000x reference)
import jax
import jax.numpy as jnp
from jax.experimental import pallas as pl


def kernel(word1, word2, W1, W2):
    raise NotImplementedError("write your pallas kernel here")


# speedup vs baseline: 1.3178x; 1.3178x over previous
import jax
import jax.numpy as jnp
from jax import lax
from jax.experimental import pallas as pl
from jax.experimental.pallas import tpu as pltpu
from jax.experimental.pallas import tpu_sc as plsc

NC = 2
NS = 16
NW = NC * NS

B = 16384
D = 128
CH = 128
BPW = B // NW
NCHUNK = BPW // CH


def _body(idx1_hbm, idx2_hbm, w1_hbm, w2_hbm, out1_hbm, out2_hbm,
          idx1_v, idx2_v, rows_a, rows_b, sem):
    wid = lax.axis_index("s") * NC + lax.axis_index("c")
    base = wid * BPW

    pltpu.sync_copy(idx1_hbm.at[wid], idx1_v)
    pltpu.sync_copy(idx2_hbm.at[wid], idx2_v)

    bufs = (rows_a, rows_b)
    for t, (idx_v, w_hbm, out_hbm) in enumerate(
            ((idx1_v, w1_hbm, out1_hbm), (idx2_v, w2_hbm, out2_hbm))):
        for j in range(NCHUNK):
            buf = bufs[j % 2]
            pltpu.async_copy(w_hbm.at[idx_v.at[j]], buf, sem).wait()
            pltpu.sync_copy(buf, out_hbm.at[pl.ds(base + j * CH, CH)])


def kernel(word1, word2, W1, W2):
    idx1 = word1.astype(jnp.int32).reshape(NW, NCHUNK, CH)
    idx2 = word2.astype(jnp.int32).reshape(NW, NCHUNK, CH)

    mesh = plsc.VectorSubcoreMesh(core_axis_name="c", subcore_axis_name="s")
    out1, out2 = pl.kernel(
        _body,
        out_type=(
            jax.ShapeDtypeStruct((B, D), jnp.float32),
            jax.ShapeDtypeStruct((B, D), jnp.float32),
        ),
        mesh=mesh,
        scratch_types=[
            pltpu.VMEM((NCHUNK, CH), jnp.int32),
            pltpu.VMEM((NCHUNK, CH), jnp.int32),
            pltpu.VMEM((CH, D), jnp.float32),
            pltpu.VMEM((CH, D), jnp.float32),
            pltpu.SemaphoreType.DMA,
        ],
    )(idx1, idx2, W1, W2)
    return (out1, out2)


# speedup vs baseline: 1.4969x; 1.1360x over previous
import jax
import jax.numpy as jnp
from jax import lax
from jax.experimental import pallas as pl
from jax.experimental.pallas import tpu as pltpu
from jax.experimental.pallas import tpu_sc as plsc

NC = 2
NS = 16
NW = NC * NS

B = 16384
D = 128
CH = 128
BPW = B // NW
NCHUNK = BPW // CH
NBUF = 7
NTOT = 2 * NCHUNK


def _body(idx1_hbm, idx2_hbm, w1_hbm, w2_hbm, out1_hbm, out2_hbm,
          idx1_v, idx2_v, *bufs_and_sems):
    bufs = bufs_and_sems[:NBUF]
    sems = bufs_and_sems[NBUF:]
    wid = lax.axis_index("s") * NC + lax.axis_index("c")
    base = wid * BPW

    pltpu.sync_copy(idx1_hbm.at[wid], idx1_v)
    pltpu.sync_copy(idx2_hbm.at[wid], idx2_v)

    chunks = [(idx1_v, w1_hbm, out1_hbm, j) for j in range(NCHUNK)] + \
             [(idx2_v, w2_hbm, out2_hbm, j) for j in range(NCHUNK)]

    def fire(c, b):
        idx_v, w_hbm, _, j = chunks[c]
        return pltpu.async_copy(w_hbm.at[idx_v.at[j]], bufs[b], sems[b])

    descs = [fire(c, c) for c in range(NBUF)]
    for c in range(NBUF):
        _, _, out_hbm, j = chunks[c]
        descs[c].wait()
        pltpu.sync_copy(bufs[c], out_hbm.at[pl.ds(base + j * CH, CH)])

    for c in range(NBUF, NTOT):
        b = c - NBUF
        _, _, out_hbm, j = chunks[c]
        fire(c, b).wait()
        pltpu.sync_copy(bufs[b], out_hbm.at[pl.ds(base + j * CH, CH)])


def kernel(word1, word2, W1, W2):
    idx1 = word1.astype(jnp.int32).reshape(NW, NCHUNK, CH)
    idx2 = word2.astype(jnp.int32).reshape(NW, NCHUNK, CH)

    mesh = plsc.VectorSubcoreMesh(core_axis_name="c", subcore_axis_name="s")
    out1, out2 = pl.kernel(
        _body,
        out_type=(
            jax.ShapeDtypeStruct((B, D), jnp.float32),
            jax.ShapeDtypeStruct((B, D), jnp.float32),
        ),
        mesh=mesh,
        scratch_types=(
            [pltpu.VMEM((NCHUNK, CH), jnp.int32)] * 2
            + [pltpu.VMEM((CH, D), jnp.float32) for _ in range(NBUF)]
            + [pltpu.SemaphoreType.DMA for _ in range(NBUF)]
        ),
    )(idx1, idx2, W1, W2)
    return (out1, out2)


# speedup vs baseline: 1.5190x; 1.0147x over previous
import jax
import jax.numpy as jnp
from jax import lax
from jax.experimental import pallas as pl
from jax.experimental.pallas import tpu as pltpu
from jax.experimental.pallas import tpu_sc as plsc

NC = 2
NS = 16
NW = NC * NS

B = 16384
D = 128
CH = 128
BPW = B // NW
NCHUNK = BPW // CH
NBUF = 7
NTOT = 2 * NCHUNK


def _body(idx1_hbm, idx2_hbm, w1_hbm, w2_hbm, out1_hbm, out2_hbm,
          idx1_v, idx2_v, *bufs_and_sems):
    bufs = bufs_and_sems[:NBUF]
    gsems = bufs_and_sems[NBUF:2 * NBUF]
    ssems = bufs_and_sems[2 * NBUF:]
    wid = lax.axis_index("s") * NC + lax.axis_index("c")
    base = wid * BPW

    pltpu.sync_copy(idx1_hbm.at[wid], idx1_v)
    pltpu.sync_copy(idx2_hbm.at[wid], idx2_v)

    chunks = [(idx1_v, w1_hbm, out1_hbm, j) for j in range(NCHUNK)] + \
             [(idx2_v, w2_hbm, out2_hbm, j) for j in range(NCHUNK)]

    def fire(c, b):
        idx_v, w_hbm, _, j = chunks[c]
        return pltpu.async_copy(w_hbm.at[idx_v.at[j]], bufs[b], gsems[b])

    def store(c, b):
        _, _, out_hbm, j = chunks[c]
        return pltpu.async_copy(
            bufs[b], out_hbm.at[pl.ds(base + j * CH, CH)], ssems[b])

    gds = [fire(c, c) for c in range(NBUF)]
    sds = []
    for c in range(NBUF):
        gds[c].wait()
        sds.append(store(c, c))
    for c in range(NBUF, NTOT):
        b = c - NBUF
        sds[b].wait()
        fire(c, b).wait()
        sds[b] = store(c, b)
    for d in sds:
        d.wait()


def kernel(word1, word2, W1, W2):
    idx1 = word1.astype(jnp.int32).reshape(NW, NCHUNK, CH)
    idx2 = word2.astype(jnp.int32).reshape(NW, NCHUNK, CH)

    mesh = plsc.VectorSubcoreMesh(core_axis_name="c", subcore_axis_name="s")
    out1, out2 = pl.kernel(
        _body,
        out_type=(
            jax.ShapeDtypeStruct((B, D), jnp.float32),
            jax.ShapeDtypeStruct((B, D), jnp.float32),
        ),
        mesh=mesh,
        scratch_types=(
            [pltpu.VMEM((NCHUNK, CH), jnp.int32)] * 2
            + [pltpu.VMEM((CH, D), jnp.float32) for _ in range(NBUF)]
            + [pltpu.SemaphoreType.DMA for _ in range(2 * NBUF)]
        ),
    )(idx1, idx2, W1, W2)
    return (out1, out2)
